# baseline (speedup 1.0000x reference)
import functools

import jax
import jax.numpy as jnp
from jax import lax
from jax.experimental import pallas as pl
from jax.experimental.pallas import tpu as pltpu
from jax.experimental.pallas import tpu_sc as plsc

N_NODES = 10000
N_EDGES = 320000
D = 128

CHUNK = 128
NC = 2
NS = 16
NW = NC * NS
EPW = N_EDGES // NW
NCH = EPW // CHUNK
TAIL = EPW - NCH * CHUNK
NBUF = 3
NQ = NCH // NBUF
N_PAD = 10112
ROWS_PER_TILE = N_PAD // NS


@functools.partial(
    pl.kernel,
    out_type=jax.ShapeDtypeStruct((NC, N_PAD, D), jnp.float32),
    mesh=plsc.VectorSubcoreMesh(core_axis_name="c", subcore_axis_name="s"),
    scratch_types=[
        pltpu.VMEM((CHUNK,), jnp.int32),
        pltpu.VMEM((CHUNK,), jnp.int32),
        pltpu.VMEM((CHUNK,), jnp.int32),
        pltpu.VMEM((CHUNK,), jnp.int32),
        pltpu.VMEM((CHUNK,), jnp.int32),
        pltpu.VMEM((CHUNK,), jnp.int32),
        pltpu.VMEM((TAIL,), jnp.int32),
        pltpu.VMEM((CHUNK, D), jnp.float32),
        pltpu.VMEM((CHUNK, D), jnp.float32),
        pltpu.VMEM((CHUNK, D), jnp.float32),
        pltpu.VMEM_SHARED((N_PAD, D), jnp.float32),
        pltpu.SemaphoreType.DMA,
        pltpu.SemaphoreType.DMA,
        pltpu.SemaphoreType.DMA,
        pltpu.SemaphoreType.DMA,
        pltpu.SemaphoreType.DMA,
        pltpu.SemaphoreType.DMA,
        pltpu.SemaphoreType.DMA,
        pltpu.SemaphoreType.DMA,
        pltpu.SemaphoreType.DMA,
    ],
)
def _sc_aggregate(src_hbm, dst_hbm, h_hbm, out_hbm,
                  srcA, srcB, srcC, dstA, dstB, dstC, dstt,
                  rows0, rows1, rows2, acc_sh,
                  g0, g1, g2, e0, e1, e2, d0, d1, d2):
    cid = lax.axis_index("c")
    sid = lax.axis_index("s")
    wid = sid * NC + cid
    ebase = wid * EPW

    rows = (rows0, rows1, rows2)
    srcs = (srcA, srcB, srcC)
    dsts = (dstA, dstB, dstC)
    gsem = (g0, g1, g2)
    esem = (e0, e1, e2)
    dsem = (d0, d1, d2)

    zvec = jnp.zeros((16,), jnp.float32)

    def zfill(r, carry):
        for q in range(8):
            rows0[r, pl.ds(q * 16, 16)] = zvec
        return carry

    lax.fori_loop(0, CHUNK, zfill, 0)

    def zcopy(k, carry):
        pltpu.sync_copy(rows0, acc_sh.at[pl.ds(sid * ROWS_PER_TILE + k * CHUNK, CHUNK)])
        return carry

    lax.fori_loop(0, 4, zcopy, 0)
    pltpu.sync_copy(
        rows0.at[pl.ds(0, ROWS_PER_TILE - 4 * CHUNK)],
        acc_sh.at[pl.ds(sid * ROWS_PER_TILE + 4 * CHUNK, ROWS_PER_TILE - 4 * CHUNK)],
    )
    plsc.subcore_barrier()

    def fire_idx(c, b):
        pltpu.async_copy(src_hbm.at[pl.ds(ebase + c * CHUNK, CHUNK)], srcs[b], esem[b])
        pltpu.async_copy(dst_hbm.at[pl.ds(ebase + c * CHUNK, CHUNK)], dsts[b], dsem[b])

    def wait_idx(b):
        pltpu.make_async_copy(src_hbm.at[pl.ds(ebase, CHUNK)], srcs[b], esem[b]).wait()
        pltpu.make_async_copy(dst_hbm.at[pl.ds(ebase, CHUNK)], dsts[b], dsem[b]).wait()

    def fire_gather(b):
        pltpu.async_copy(h_hbm.at[srcs[b]], rows[b], gsem[b])

    def wait_gather(b):
        pltpu.make_async_copy(h_hbm.at[srcs[b]], rows[b], gsem[b]).wait()

    def fire_scatter(b):
        pltpu.sync_copy(rows[b], acc_sh.at[dsts[b]], add=True)

    for b in range(3):
        fire_idx(b, b)
    for b in range(2):
        wait_idx(b)
        fire_gather(b)

    def body(j, carry):
        for p in range(NBUF):
            c = NBUF * j + p
            pn = (p + 2) % NBUF

            @pl.when(c + 2 < NCH)
            def _():
                wait_idx(pn)
                fire_gather(pn)

            wait_gather(p)
            fire_scatter(p)

            @pl.when(c + 3 < NCH)
            def _():
                fire_idx(c + 3, p)

        return carry

    lax.fori_loop(0, NQ, body, 0)

    pltpu.sync_copy(dst_hbm.at[pl.ds(ebase + NCH * CHUNK, TAIL)], dstt)
    pltpu.sync_copy(src_hbm.at[pl.ds(ebase + NCH * CHUNK, TAIL)],
                    srcA.at[pl.ds(0, TAIL)])
    pltpu.async_copy(
        h_hbm.at[srcA.at[pl.ds(0, TAIL)]], rows0.at[pl.ds(0, TAIL)], g0
    ).wait()
    pltpu.sync_copy(rows0.at[pl.ds(0, TAIL)], acc_sh.at[dstt], add=True)

    plsc.subcore_barrier()

    pltpu.sync_copy(
        acc_sh.at[pl.ds(sid * ROWS_PER_TILE, ROWS_PER_TILE)],
        out_hbm.at[cid, pl.ds(sid * ROWS_PER_TILE, ROWS_PER_TILE)],
    )


BLK = 2000


def _mlp_body(scale_ref, h_ref, p_ref, w1_ref, b1_ref, w2_ref, b2_ref, o_ref):
    scale = scale_ref[0]
    z = h_ref[...] * scale + p_ref[0] + p_ref[1]
    z = jnp.dot(z, w1_ref[...], preferred_element_type=jnp.float32) + b1_ref[...]
    z = jnp.maximum(z, 0.0)
    z = jnp.dot(z, w2_ref[...], preferred_element_type=jnp.float32) + b2_ref[...]
    o_ref[...] = jnp.maximum(z, 0.0)


_tc_mlp = pl.pallas_call(
    _mlp_body,
    grid=(N_NODES // BLK,),
    in_specs=[
        pl.BlockSpec(memory_space=pltpu.SMEM),
        pl.BlockSpec((BLK, D), lambda i: (i, 0)),
        pl.BlockSpec((NC, BLK, D), lambda i: (0, i, 0)),
        pl.BlockSpec((D, D), lambda i: (0, 0)),
        pl.BlockSpec((1, D), lambda i: (0, 0)),
        pl.BlockSpec((D, D), lambda i: (0, 0)),
        pl.BlockSpec((1, D), lambda i: (0, 0)),
    ],
    out_specs=pl.BlockSpec((BLK, D), lambda i: (i, 0)),
    out_shape=jax.ShapeDtypeStruct((N_NODES, D), jnp.float32),
)


def _gin_layer(h, src, dst, eps, W1, b1, W2, b2):
    parts = _sc_aggregate(src, dst, h)
    scale = (1.0 + eps).reshape((1,)).astype(jnp.float32)
    return _tc_mlp(scale, h, parts, W1, b1.reshape(1, D), W2, b2.reshape(1, D))


def kernel(x, edge_index, eps0, W1_0, b1_0, W2_0, b2_0, eps1, W1_1, b1_1, W2_1, b2_1):
    src = edge_index[0]
    dst = edge_index[1]
    h = _gin_layer(x, src, dst, eps0, W1_0, b1_0, W2_0, b2_0)
    h = _gin_layer(h, src, dst, eps1, W1_1, b1_1, W2_1, b2_1)
    return h


# speedup vs baseline: 1.1087x; 1.1087x over previous
import functools

import jax
import jax.numpy as jnp
from jax import lax
from jax.experimental import pallas as pl
from jax.experimental.pallas import tpu as pltpu
from jax.experimental.pallas import tpu_sc as plsc

N_NODES = 10000
N_EDGES = 320000
D = 128

CHUNK = 128
NC = 2
NS = 16
NW = NC * NS
EPW = N_EDGES // NW
NCH = EPW // CHUNK
TAIL = EPW - NCH * CHUNK
N_PAD = 10112
ROWS_PER_TILE = N_PAD // NS
NZ = 5


@functools.partial(
    pl.kernel,
    out_type=jax.ShapeDtypeStruct((NC, N_PAD, D), jnp.float32),
    mesh=plsc.VectorSubcoreMesh(core_axis_name="c", subcore_axis_name="s"),
    scratch_types=[
        pltpu.VMEM((EPW,), jnp.int32),
        pltpu.VMEM((CHUNK,), jnp.int32),
        pltpu.VMEM((CHUNK,), jnp.int32),
        pltpu.VMEM((TAIL,), jnp.int32),
        pltpu.VMEM((CHUNK, D), jnp.float32),
        pltpu.VMEM((CHUNK, D), jnp.float32),
        pltpu.VMEM_SHARED((N_PAD, D), jnp.float32),
        pltpu.SemaphoreType.DMA,
        pltpu.SemaphoreType.DMA,
        pltpu.SemaphoreType.DMA,
        pltpu.SemaphoreType.DMA,
        pltpu.SemaphoreType.DMA,
    ],
)
def _sc_aggregate(src_hbm, dst_hbm, h_hbm, out_hbm,
                  src_v, dst0, dst1, dstt, rows0, rows1, acc_sh,
                  gsem0, gsem1, dsem0, dsem1, zsem):
    cid = lax.axis_index("c")
    sid = lax.axis_index("s")
    wid = sid * NC + cid
    ebase = wid * EPW

    zvec = jnp.zeros((16,), jnp.float32)

    def zfill(r, carry):
        for q in range(8):
            rows1[r, pl.ds(q * 16, 16)] = zvec
        return carry

    lax.fori_loop(0, CHUNK, zfill, 0)

    for k in range(NZ - 1):
        pltpu.async_copy(
            rows1, acc_sh.at[pl.ds(sid * ROWS_PER_TILE + k * CHUNK, CHUNK)], zsem)
    last = ROWS_PER_TILE - (NZ - 1) * CHUNK
    pltpu.async_copy(
        rows1.at[pl.ds(0, last)],
        acc_sh.at[pl.ds(sid * ROWS_PER_TILE + (NZ - 1) * CHUNK, last)], zsem)

    pltpu.sync_copy(src_hbm.at[pl.ds(ebase, EPW)], src_v)

    dsts = (dst0, dst1)
    dsems = (dsem0, dsem1)
    rows = (rows0, rows1)
    gsems = (gsem0, gsem1)

    def fire_dst(c, b):
        pltpu.async_copy(dst_hbm.at[pl.ds(ebase + c * CHUNK, CHUNK)], dsts[b], dsems[b])

    def fire_gather(c, b):
        pltpu.async_copy(h_hbm.at[src_v.at[pl.ds(c * CHUNK, CHUNK)]], rows[b], gsems[b])

    def wait_dst(c, b):
        pltpu.make_async_copy(
            dst_hbm.at[pl.ds(ebase + c * CHUNK, CHUNK)], dsts[b], dsems[b]).wait()

    def wait_gather(c, b):
        pltpu.make_async_copy(
            h_hbm.at[src_v.at[pl.ds(c * CHUNK, CHUNK)]], rows[b], gsems[b]).wait()

    fire_dst(0, 0)
    fire_dst(1, 1)
    fire_gather(0, 0)

    for k in range(NZ - 1):
        pltpu.make_async_copy(
            rows1, acc_sh.at[pl.ds(sid * ROWS_PER_TILE, CHUNK)], zsem).wait()
    pltpu.make_async_copy(
        rows1.at[pl.ds(0, last)],
        acc_sh.at[pl.ds(sid * ROWS_PER_TILE, last)], zsem).wait()
    plsc.subcore_barrier()

    def body(j, carry):
        c0 = 2 * j
        c1 = 2 * j + 1
        fire_gather(c1, 1)
        wait_gather(c0, 0)
        wait_dst(c0, 0)
        pltpu.sync_copy(rows0, acc_sh.at[dst0], add=True)

        @pl.when(c1 + 1 < NCH)
        def _():
            fire_dst(c0 + 2, 0)
            fire_gather(c1 + 1, 0)

        wait_gather(c1, 1)
        wait_dst(c1, 1)
        pltpu.sync_copy(rows1, acc_sh.at[dst1], add=True)

        @pl.when(c1 + 2 < NCH)
        def _():
            fire_dst(c1 + 2, 1)

        return carry

    lax.fori_loop(0, NCH // 2, body, 0)

    pltpu.sync_copy(dst_hbm.at[pl.ds(ebase + NCH * CHUNK, TAIL)], dstt)
    pltpu.async_copy(
        h_hbm.at[src_v.at[pl.ds(NCH * CHUNK, TAIL)]], rows0.at[pl.ds(0, TAIL)], gsem0
    ).wait()
    pltpu.sync_copy(rows0.at[pl.ds(0, TAIL)], acc_sh.at[dstt], add=True)

    plsc.subcore_barrier()

    pltpu.sync_copy(
        acc_sh.at[pl.ds(sid * ROWS_PER_TILE, ROWS_PER_TILE)],
        out_hbm.at[cid, pl.ds(sid * ROWS_PER_TILE, ROWS_PER_TILE)],
    )


BLK = 2000


def _mlp_body(scale_ref, h_ref, p_ref, w1_ref, b1_ref, w2_ref, b2_ref, o_ref):
    scale = scale_ref[0]
    z = h_ref[...] * scale + p_ref[0] + p_ref[1]
    z = jnp.dot(z, w1_ref[...], preferred_element_type=jnp.float32) + b1_ref[...]
    z = jnp.maximum(z, 0.0)
    z = jnp.dot(z, w2_ref[...], preferred_element_type=jnp.float32) + b2_ref[...]
    o_ref[...] = jnp.maximum(z, 0.0)


_tc_mlp = pl.pallas_call(
    _mlp_body,
    grid=(N_NODES // BLK,),
    in_specs=[
        pl.BlockSpec(memory_space=pltpu.SMEM),
        pl.BlockSpec((BLK, D), lambda i: (i, 0)),
        pl.BlockSpec((NC, BLK, D), lambda i: (0, i, 0)),
        pl.BlockSpec((D, D), lambda i: (0, 0)),
        pl.BlockSpec((1, D), lambda i: (0, 0)),
        pl.BlockSpec((D, D), lambda i: (0, 0)),
        pl.BlockSpec((1, D), lambda i: (0, 0)),
    ],
    out_specs=pl.BlockSpec((BLK, D), lambda i: (i, 0)),
    out_shape=jax.ShapeDtypeStruct((N_NODES, D), jnp.float32),
)


def _gin_layer(h, src, dst, eps, W1, b1, W2, b2):
    parts = _sc_aggregate(src, dst, h)
    scale = (1.0 + eps).reshape((1,)).astype(jnp.float32)
    return _tc_mlp(scale, h, parts, W1, b1.reshape(1, D), W2, b2.reshape(1, D))


def kernel(x, edge_index, eps0, W1_0, b1_0, W2_0, b2_0, eps1, W1_1, b1_1, W2_1, b2_1):
    src = edge_index[0]
    dst = edge_index[1]
    h = _gin_layer(x, src, dst, eps0, W1_0, b1_0, W2_0, b2_0)
    h = _gin_layer(h, src, dst, eps1, W1_1, b1_1, W2_1, b2_1)
    return h
